# Initial kernel scaffold; baseline (speedup 1.0000x reference)
import jax
import jax.numpy as jnp
from jax.experimental import pallas as pl


def kernel(in_nodes_features, edge_index, W_proj, a_src, a_trg, W_skip):
    raise NotImplementedError("write your pallas kernel here")


# speedup vs baseline: 31.0793x; 31.0793x over previous
import functools

import jax
import jax.numpy as jnp
from jax import lax
from jax.experimental import pallas as pl
from jax.experimental.pallas import tpu as pltpu
from jax.experimental.pallas import tpu_sc as plsc

_N = 10000
_E = 320000
_DIN = 128
_H = 8
_F = 16

_BN = 1000
_CH = 80
_EPT = _E // 4
_STG = 20000
_NST = _EPT // _STG
_NCH = _STG // _CH


def _pre_body(x_ref, wp_ref, ws_ref, avs_ref, avt_ref,
              proj_ref, ssrc_ref, strg_ref, skip_ref, ms_ref, mt_ref):
    i = pl.program_id(0)
    x = x_ref[...]
    p = jnp.dot(x, wp_ref[...], preferred_element_type=jnp.float32)
    skip_ref[...] = jnp.dot(x, ws_ref[...], preferred_element_type=jnp.float32)
    grp = (lax.broadcasted_iota(jnp.int32, (_DIN, _H), 0) // _F
           == lax.broadcasted_iota(jnp.int32, (_DIN, _H), 1)).astype(jnp.float32)
    ssrc = jnp.dot(p * avs_ref[...], grp, preferred_element_type=jnp.float32)
    strg = jnp.dot(p * avt_ref[...], grp, preferred_element_type=jnp.float32)
    ssrc_ref[...] = ssrc
    strg_ref[...] = strg
    for h in range(_H):
        proj_ref[h] = p[:, h * _F:(h + 1) * _F]
    bs = jnp.max(ssrc, axis=0, keepdims=True)
    bt = jnp.max(strg, axis=0, keepdims=True)

    @pl.when(i == 0)
    def _():
        ms_ref[...] = bs
        mt_ref[...] = bt

    @pl.when(i > 0)
    def _():
        ms_ref[...] = jnp.maximum(ms_ref[...], bs)
        mt_ref[...] = jnp.maximum(mt_ref[...], bt)


def _tc_pre(x, w_proj, w_skip, avs, avt):
    nb = _N // _BN
    return pl.pallas_call(
        _pre_body,
        grid=(nb,),
        in_specs=[
            pl.BlockSpec((_BN, _DIN), lambda i: (i, 0)),
            pl.BlockSpec((_DIN, _DIN), lambda i: (0, 0)),
            pl.BlockSpec((_DIN, _DIN), lambda i: (0, 0)),
            pl.BlockSpec((1, _DIN), lambda i: (0, 0)),
            pl.BlockSpec((1, _DIN), lambda i: (0, 0)),
        ],
        out_specs=[
            pl.BlockSpec((_H, _BN, _F), lambda i: (0, i, 0)),
            pl.BlockSpec((_BN, _H), lambda i: (i, 0)),
            pl.BlockSpec((_BN, _H), lambda i: (i, 0)),
            pl.BlockSpec((_BN, _DIN), lambda i: (i, 0)),
            pl.BlockSpec((1, _H), lambda i: (0, 0)),
            pl.BlockSpec((1, _H), lambda i: (0, 0)),
        ],
        out_shape=[
            jax.ShapeDtypeStruct((_H, _N, _F), jnp.float32),
            jax.ShapeDtypeStruct((_N, _H), jnp.float32),
            jax.ShapeDtypeStruct((_N, _H), jnp.float32),
            jax.ShapeDtypeStruct((_N, _DIN), jnp.float32),
            jax.ShapeDtypeStruct((1, _H), jnp.float32),
            jax.ShapeDtypeStruct((1, _H), jnp.float32),
        ],
    )(x, w_proj, w_skip, avs, avt)


def _sc_body(ssrcT, strgT, dvec, epk, projf,
             out0, out1, dens,
             ssrc_v, strg_v, dv_v, pkb, gidx_v, widx_v, didx_v,
             ebuf, pbuf, dbuf, zbuf, out_sh, den_pk, sem_g):
    c = lax.axis_index("c")
    s = lax.axis_index("s")
    hl = s // 4
    q = s % 4
    h = c * 4 + hl

    pltpu.sync_copy(ssrcT.at[pl.ds(h * _N, _N)], ssrc_v)
    pltpu.sync_copy(strgT.at[pl.ds(h * _N, _N)], strg_v)
    pltpu.sync_copy(dvec.at[pl.ds(h * _F, _F)], dv_v)
    dval = dv_v[...]

    def _z(j, carry):
        zbuf[j, :] = jnp.zeros((16,), jnp.float32)
        return carry
    lax.fori_loop(0, 1000, _z, 0)

    @pl.when(s % 2 == 0)
    def _():
        for k in range(5):
            zsl = pl.ds((s // 2) * 5000 + k * 1000, 1000)
            pltpu.sync_copy(zbuf, out_sh.at[zsl])

    @pl.when(s < 5)
    def _():
        for k in range(2):
            pltpu.sync_copy(zbuf, den_pk.at[pl.ds(s * 2000 + k * 1000, 1000)])
    plsc.subcore_barrier()

    hmask = jnp.where(lax.iota(jnp.int32, 16) == hl, 1.0, 0.0)

    hbase = h * _N

    def _stage(stg):
        ebase = q * _EPT + stg * _STG
        pltpu.sync_copy(epk.at[pl.ds(ebase, _STG)], pkb)

        def _chunk(i, carry):
            off = i * _CH
            for g in range(_CH // 16):
                sl = pl.ds(g * 16, 16)
                pk = pkb[pl.ds(off + g * 16, 16)]
                sv = lax.shift_right_logical(pk, 14)
                tv = pk & 16383
                gidx_v[sl] = sv + hbase
                widx_v[sl] = tv * 4 + hl
                didx_v[sl] = tv
            pltpu.async_copy(projf.at[gidx_v], pbuf, sem_g).wait()
            for g in range(_CH // 16):
                pk = pkb[pl.ds(off + g * 16, 16)]
                sv = lax.shift_right_logical(pk, 14)
                tv = pk & 16383
                a = plsc.load_gather(ssrc_v, [sv])
                b = plsc.load_gather(strg_v, [tv])
                u = a + b
                sc = jnp.maximum(u, 0.2 * u) - dval
                e = jnp.exp(sc)
                for r in range(16):
                    row = g * 16 + r
                    es = e.at[jnp.full((16,), r, jnp.int32)].get(
                        mode='promise_in_bounds')
                    pbuf[row, :] = pbuf[row, :] * es
                    dbuf[row, :] = es * hmask
            pltpu.sync_copy(pbuf, out_sh.at[widx_v], add=True)
            pltpu.sync_copy(dbuf, den_pk.at[didx_v], add=True)
            return carry
        lax.fori_loop(0, _NCH, _chunk, 0)

    for stg in range(_NST):
        _stage(stg)

    plsc.subcore_barrier()

    @pl.when(s < 5)
    def _():
        pltpu.sync_copy(den_pk.at[pl.ds(s * 2000, 2000)],
                        dens.at[pl.ds(c * _N + s * 2000, 2000)])

    @pl.when(s % 2 == 0)
    def _():
        sl = pl.ds((s // 2) * 5000, 5000)

        @pl.when(c == 0)
        def _():
            pltpu.sync_copy(out_sh.at[sl], out0.at[sl])

        @pl.when(c == 1)
        def _():
            pltpu.sync_copy(out_sh.at[sl], out1.at[sl])


def _sc_edge(ssrcT, strgT, dvec, epk, projf):
    mesh = plsc.VectorSubcoreMesh(core_axis_name="c", subcore_axis_name="s")
    acc = jax.ShapeDtypeStruct((_N * 4, _F), jnp.float32)
    kern = pl.kernel(
        _sc_body,
        out_type=(acc, acc, jax.ShapeDtypeStruct((2 * _N, _F), jnp.float32)),
        mesh=mesh,
        compiler_params=pltpu.CompilerParams(
            needs_layout_passes=False, use_tc_tiling_on_sc=False),
        scratch_types=[
            pltpu.VMEM((_N,), jnp.float32),
            pltpu.VMEM((_N,), jnp.float32),
            pltpu.VMEM((16,), jnp.float32),
            pltpu.VMEM((_STG,), jnp.int32),
            pltpu.VMEM((_CH,), jnp.int32),
            pltpu.VMEM((_CH,), jnp.int32),
            pltpu.VMEM((_CH,), jnp.int32),
            pltpu.VMEM((_CH,), jnp.float32),
            pltpu.VMEM((_CH, _F), jnp.float32),
            pltpu.VMEM((_CH, _F), jnp.float32),
            pltpu.VMEM((1000, _F), jnp.float32),
            pltpu.VMEM_SHARED((_N * 4, _F), jnp.float32),
            pltpu.VMEM_SHARED((_N, _F), jnp.float32),
            pltpu.SemaphoreType.DMA,
        ],
    )
    return kern(ssrcT, strgT, dvec, epk, projf)


def _post_body(u_ref, d_ref, skip_ref, o_ref):
    o = u_ref[...] / (d_ref[...] + 1e-16) + skip_ref[...]
    o_ref[...] = jnp.where(o > 0, o, jnp.exp(jnp.minimum(o, 0.0)) - 1.0)


def _tc_post(ucat, drep, skip):
    nb = _N // _BN
    full = pl.BlockSpec((_BN, _DIN), lambda i: (i, 0))
    return pl.pallas_call(
        _post_body,
        grid=(nb,),
        in_specs=[full, full, full],
        out_specs=full,
        out_shape=jax.ShapeDtypeStruct((_N, _DIN), jnp.float32),
    )(ucat, drep, skip)


def kernel(in_nodes_features, edge_index, W_proj, a_src, a_trg, W_skip):
    x = in_nodes_features
    avs = a_src.reshape(1, _H * _F)
    avt = a_trg.reshape(1, _H * _F)
    proj, ssrc, strg, skip, ms, mt = _tc_pre(x, W_proj, W_skip, avs, avt)
    cc = ms + mt
    d = jnp.where(cc > 0, cc, 0.2 * cc)
    dvec = jnp.broadcast_to(d.reshape(_H, 1), (_H, _F))
    epk = (edge_index[0] << 14) | edge_index[1]
    out0, out1, dens = _sc_edge(
        ssrc.T.reshape(-1), strg.T.reshape(-1), dvec.reshape(-1),
        epk, proj.reshape(_H * _N, _F))
    ucat = jnp.concatenate(
        [out0.reshape(_N, 4 * _F), out1.reshape(_N, 4 * _F)], axis=-1)
    dp = dens.reshape(2, _N, _F)
    denT = jnp.concatenate([dp[0][:, 0:4], dp[1][:, 0:4]], axis=1)
    drep = jnp.repeat(denT, _F, axis=1)
    return _tc_post(ucat, drep, skip)
